# baseline (speedup 1.0000x reference)
import functools

import jax
import jax.numpy as jnp
from jax import lax
from jax.experimental import pallas as pl
from jax.experimental.pallas import tpu as pltpu
from jax.experimental.pallas import tpu_sc as plsc

SCALE_ = 8.0
_BAND = 128
_BANDS_PER_CHUNK = 2
_CHUNK = _BAND * _BANDS_PER_CHUNK
_D = 64


def _make_emb(n_bands: int, seq: int, vocab: int):
  info = plsc.get_sparse_core_info()
  nc, ns, nl = info.num_cores, info.num_subcores, info.num_lanes
  nw = nc * ns
  bands_w = n_bands // nw
  n_chunks = bands_w // _BANDS_PER_CHUNK
  tjn = n_bands // seq
  assert bands_w % (2 * _BANDS_PER_CHUNK) == 0 and nl == 16 and _D == 64

  mesh = plsc.VectorSubcoreMesh(core_axis_name="c", subcore_axis_name="s")

  @functools.partial(
      pl.kernel,
      mesh=mesh,
      compiler_params=pltpu.CompilerParams(use_tc_tiling_on_sc=False,
                                           needs_layout_passes=False,
                                           disable_bounds_checks=True),
      out_type=jax.ShapeDtypeStruct((seq, 8, tjn, 8, _BAND), jnp.float32),
      scratch_types=[
          pltpu.VMEM((bands_w, _BAND), jnp.int32),
          pltpu.VMEM((_CHUNK, _D), jnp.float32),
          pltpu.VMEM((_CHUNK, _D), jnp.float32),
          pltpu.VMEM((8, _BANDS_PER_CHUNK, 8, _BAND), jnp.float32),
          pltpu.VMEM((8, _BANDS_PER_CHUNK, 8, _BAND), jnp.float32),
          pltpu.SemaphoreType.DMA,
          pltpu.SemaphoreType.DMA,
          pltpu.SemaphoreType.DMA,
          pltpu.SemaphoreType.DMA,
      ],
  )
  def emb(idx_hbm, table_hbm, out_hbm, idx_v, g0, g1, t0, t1,
          sem_g0, sem_g1, sem_w0, sem_w1):
    wid = lax.axis_index("s") * nc + lax.axis_index("c")
    band0 = wid * bands_w
    bufs = ((g0, t0, sem_g0, sem_w0), (g1, t1, sem_g1, sem_w1))
    iota = lax.iota(jnp.int32, nl)

    pltpu.sync_copy(idx_hbm.at[pl.ds(band0, bands_w)], idx_v)

    def fire(q, g_v, sem):
      for j in range(_BANDS_PER_CHUNK):
        pltpu.async_copy(
            table_hbm.at[idx_v.at[q * _BANDS_PER_CHUNK + j]],
            g_v.at[pl.ds(j * _BAND, _BAND)],
            sem,
        )

    def drain_g(g_v, sem):
      pltpu.make_async_copy(table_hbm.at[pl.ds(0, _CHUNK)], g_v, sem).wait()

    def drain_w(t_v, sem):
      pltpu.make_async_copy(
          out_hbm.at[0, :, pl.ds(0, _BANDS_PER_CHUNK)], t_v, sem).wait()

    ridxs = tuple(
        tuple(iota + (band * _BAND + g * nl) for g in range(_BAND // nl))
        for band in range(_BANDS_PER_CHUNK))

    def transpose_scale(g_v, t_v):
      def jstep(j, carry):
        jvec = jnp.broadcast_to(j, (nl,))
        for ti in range(8):
          dvec = jvec + (8 * ti)
          for band in range(_BANDS_PER_CHUNK):
            for g in range(_BAND // nl):
              val = plsc.load_gather(g_v, [ridxs[band][g], dvec])
              t_v[ti, band, j, pl.ds(g * nl, nl)] = val * SCALE_
        return carry

      lax.fori_loop(0, 8, jstep, 0)

    fire(0, g0, sem_g0)
    fire(1, g1, sem_g1)

    def chunk_pair(k, carry):
      for b, (g_v, t_v, sem_g, sem_w) in enumerate(bufs):
        q = 2 * k + b
        gb = band0 + q * _BANDS_PER_CHUNK
        s = gb // tjn
        tj = gb % tjn
        drain_g(g_v, sem_g)

        @pl.when(k > 0)
        def _tr_free():
          drain_w(t_v, sem_w)

        transpose_scale(g_v, t_v)

        @pl.when(k < n_chunks // 2 - 1)
        def _prefetch():
          fire(q + 2, g_v, sem_g)

        pltpu.async_copy(
            t_v,
            out_hbm.at[s, :, pl.ds(tj, _BANDS_PER_CHUNK)],
            sem_w,
        )
      return carry

    lax.fori_loop(0, n_chunks // 2, chunk_pair, 0)
    drain_w(t0, sem_w0)
    drain_w(t1, sem_w1)

  return emb


def kernel(x, table):
  b, s = x.shape
  vocab, d = table.shape
  n_bands = (b * s) // _BAND
  tjn = b // _BAND
  idx2d = x.T.reshape(n_bands, _BAND)
  emb = _make_emb(n_bands, s, vocab)
  out5 = emb(idx2d, table)
  return out5.transpose(2, 4, 0, 1, 3).reshape(b, s, d)


# speedup vs baseline: 1.4215x; 1.4215x over previous
import functools

import jax
import jax.numpy as jnp
from jax import lax
from jax.experimental import pallas as pl
from jax.experimental.pallas import tpu as pltpu
from jax.experimental.pallas import tpu_sc as plsc

SCALE_ = 8.0
_BAND = 128
_BANDS_PER_CHUNK = 2
_CHUNK = _BAND * _BANDS_PER_CHUNK
_D = 64


def _make_emb(n_bands: int, seq: int, vocab: int):
  info = plsc.get_sparse_core_info()
  nc, ns, nl = info.num_cores, info.num_subcores, info.num_lanes
  nw = nc * ns
  bands_w = n_bands // nw
  n_chunks = bands_w // _BANDS_PER_CHUNK
  tjn = n_bands // seq
  assert bands_w % (2 * _BANDS_PER_CHUNK) == 0 and nl == 16 and _D == 64

  mesh = plsc.VectorSubcoreMesh(core_axis_name="c", subcore_axis_name="s")

  @functools.partial(
      pl.kernel,
      mesh=mesh,
      compiler_params=pltpu.CompilerParams(use_tc_tiling_on_sc=False,
                                           needs_layout_passes=False,
                                           disable_bounds_checks=True),
      out_type=jax.ShapeDtypeStruct((seq, 8, tjn, 8, _BAND), jnp.float32),
      scratch_types=[
          pltpu.VMEM((bands_w, _BAND), jnp.int32),
          pltpu.VMEM((_CHUNK, _D), jnp.float32),
          pltpu.VMEM((_CHUNK, _D), jnp.float32),
          pltpu.VMEM((8, _BANDS_PER_CHUNK, 8, _BAND), jnp.float32),
          pltpu.VMEM((8, _BANDS_PER_CHUNK, 8, _BAND), jnp.float32),
          pltpu.SemaphoreType.DMA,
          pltpu.SemaphoreType.DMA,
          pltpu.SemaphoreType.DMA,
          pltpu.SemaphoreType.DMA,
      ],
  )
  def emb(idx_hbm, table_hbm, out_hbm, idx_v, g0, g1, t0, t1,
          sem_g0, sem_g1, sem_w0, sem_w1):
    wid = lax.axis_index("s") * nc + lax.axis_index("c")
    band0 = wid * bands_w
    bufs = ((g0, t0, sem_g0, sem_w0), (g1, t1, sem_g1, sem_w1))
    iota = lax.iota(jnp.int32, nl)

    pltpu.sync_copy(idx_hbm.at[pl.ds(band0, bands_w)], idx_v)

    def fire(q, g_v, sem):
      for j in range(_BANDS_PER_CHUNK):
        pltpu.async_copy(
            table_hbm.at[idx_v.at[q * _BANDS_PER_CHUNK + j]],
            g_v.at[pl.ds(j * _BAND, _BAND)],
            sem,
        )

    def drain_g(g_v, sem):
      pltpu.make_async_copy(table_hbm.at[pl.ds(0, _CHUNK)], g_v, sem).wait()

    def drain_w(t_v, sem):
      pltpu.make_async_copy(
          out_hbm.at[0, :, pl.ds(0, _BANDS_PER_CHUNK)], t_v, sem).wait()

    ridxs = tuple(
        tuple(iota + (band * _BAND + g * nl) for g in range(_BAND // nl))
        for band in range(_BANDS_PER_CHUNK))

    def transpose_scale(g_v, t_v):
      @plsc.parallel_loop(0, 8)
      def jstep(j):
        jvec = jnp.broadcast_to(j, (nl,))
        for ti in range(8):
          dvec = jvec + (8 * ti)
          for band in range(_BANDS_PER_CHUNK):
            for g in range(_BAND // nl):
              val = plsc.load_gather(g_v, [ridxs[band][g], dvec])
              t_v[ti, band, j, pl.ds(g * nl, nl)] = val * SCALE_

    fire(0, g0, sem_g0)
    fire(1, g1, sem_g1)

    def chunk_pair(k, carry):
      for b, (g_v, t_v, sem_g, sem_w) in enumerate(bufs):
        q = 2 * k + b
        gb = band0 + q * _BANDS_PER_CHUNK
        s = gb // tjn
        tj = gb % tjn
        drain_g(g_v, sem_g)

        @pl.when(k > 0)
        def _tr_free():
          drain_w(t_v, sem_w)

        transpose_scale(g_v, t_v)

        @pl.when(k < n_chunks // 2 - 1)
        def _prefetch():
          fire(q + 2, g_v, sem_g)

        pltpu.async_copy(
            t_v,
            out_hbm.at[s, :, pl.ds(tj, _BANDS_PER_CHUNK)],
            sem_w,
        )
      return carry

    lax.fori_loop(0, n_chunks // 2, chunk_pair, 0)
    drain_w(t0, sem_w0)
    drain_w(t1, sem_w1)

  return emb


def kernel(x, table):
  b, s = x.shape
  vocab, d = table.shape
  n_bands = (b * s) // _BAND
  tjn = b // _BAND
  idx2d = x.T.reshape(n_bands, _BAND)
  emb = _make_emb(n_bands, s, vocab)
  out5 = emb(idx2d, table)
  return out5.transpose(2, 4, 0, 1, 3).reshape(b, s, d)


# speedup vs baseline: 1.4445x; 1.0162x over previous
import functools

import jax
import jax.numpy as jnp
from jax import lax
from jax.experimental import pallas as pl
from jax.experimental.pallas import tpu as pltpu
from jax.experimental.pallas import tpu_sc as plsc

SCALE_ = 8.0
_BAND = 128
_D = 64


def _make_emb(n_bands: int, seq: int):
  info = plsc.get_sparse_core_info()
  nc, ns, nl = info.num_cores, info.num_subcores, info.num_lanes
  nw = nc * ns
  bands_w = n_bands // nw
  tjn = n_bands // seq
  assert bands_w % 2 == 0 and nl == 16 and _D == 64

  mesh = plsc.VectorSubcoreMesh(core_axis_name="c", subcore_axis_name="s")

  @functools.partial(
      pl.kernel,
      mesh=mesh,
      compiler_params=pltpu.CompilerParams(use_tc_tiling_on_sc=False,
                                           needs_layout_passes=False,
                                           disable_bounds_checks=True),
      out_type=jax.ShapeDtypeStruct((seq, 8, tjn, 8, _BAND), jnp.float32),
      scratch_types=[
          pltpu.VMEM((bands_w, _BAND), jnp.int32),
          pltpu.VMEM((bands_w, _BAND), jnp.int32),
          pltpu.VMEM((_BAND, 2 * _D), jnp.float32),
          pltpu.VMEM((_BAND, 2 * _D), jnp.float32),
          pltpu.VMEM((8, 1, 8, _BAND), jnp.float32),
          pltpu.VMEM((8, 1, 8, _BAND), jnp.float32),
          pltpu.SemaphoreType.DMA,
          pltpu.SemaphoreType.DMA,
          pltpu.SemaphoreType.DMA,
          pltpu.SemaphoreType.DMA,
      ],
  )
  def emb(idx_hbm, table_hbm, out_hbm, idx_v, idx2_v, g0, g1, t0, t1,
          sem_g0, sem_g1, sem_w0, sem_w1):
    wid = lax.axis_index("s") * nc + lax.axis_index("c")
    band0 = wid * bands_w
    bufs = ((g0, t0, sem_g0, sem_w0), (g1, t1, sem_g1, sem_w1))
    iota = lax.iota(jnp.int32, nl)

    pltpu.sync_copy(idx_hbm.at[pl.ds(band0, bands_w)], idx_v)

    @plsc.parallel_loop(0, bands_w)
    def _halve(r):
      for g in range(_BAND // nl):
        sl = pl.ds(g * nl, nl)
        idx2_v[r, sl] = idx_v[r, sl] >> 1

    def fire(q, g_v, sem):
      pltpu.async_copy(table_hbm.at[idx2_v.at[q]], g_v, sem)

    def drain_g(g_v, sem):
      pltpu.make_async_copy(table_hbm.at[pl.ds(0, _BAND)], g_v, sem).wait()

    def drain_w(t_v, sem):
      pltpu.make_async_copy(out_hbm.at[0, :, pl.ds(0, 1)], t_v, sem).wait()

    ridxs = tuple(iota + g * nl for g in range(_BAND // nl))

    def transpose_scale(q, g_v, t_v):
      pvs = tuple(
          (idx_v[q, pl.ds(g * nl, nl)] & 1) << 6 for g in range(_BAND // nl))

      @plsc.parallel_loop(0, 8)
      def jstep(j):
        jvec = jnp.broadcast_to(j, (nl,))
        for g in range(_BAND // nl):
          pj = pvs[g] + jvec
          for ti in range(8):
            val = plsc.load_gather(g_v, [ridxs[g], pj + (8 * ti)])
            t_v[ti, 0, j, pl.ds(g * nl, nl)] = val * SCALE_

    fire(0, g0, sem_g0)
    fire(1, g1, sem_g1)

    def chunk_pair(k, carry):
      for b, (g_v, t_v, sem_g, sem_w) in enumerate(bufs):
        q = 2 * k + b
        gb = band0 + q
        s = gb // tjn
        tj = gb % tjn
        drain_g(g_v, sem_g)

        @pl.when(k > 0)
        def _tr_free():
          drain_w(t_v, sem_w)

        transpose_scale(q, g_v, t_v)

        @pl.when(k < bands_w // 2 - 1)
        def _prefetch():
          fire(q + 2, g_v, sem_g)

        pltpu.async_copy(t_v, out_hbm.at[s, :, pl.ds(tj, 1)], sem_w)
      return carry

    lax.fori_loop(0, bands_w // 2, chunk_pair, 0)
    drain_w(t0, sem_w0)
    drain_w(t1, sem_w1)

  return emb


def kernel(x, table):
  b, s = x.shape
  vocab, d = table.shape
  n_bands = (b * s) // _BAND
  tjn = b // _BAND
  idx2d = x.T.reshape(n_bands, _BAND)
  table2 = table.reshape(vocab // 2, 2 * d)
  emb = _make_emb(n_bands, s)
  out5 = emb(idx2d, table2)
  return out5.transpose(2, 4, 0, 1, 3).reshape(b, s, d)
